# Initial kernel scaffold; baseline (speedup 1.0000x reference)
import functools

import jax
import jax.numpy as jnp
from jax import lax
from jax.experimental import pallas as pl
from jax.experimental.pallas import tpu as pltpu
from jax.experimental.pallas import tpu_sc as plsc

N_NODES = 10000
N_EDGES = 320000
D_FEAT = 128

NC = 2
NS = 16
L = 16
NW = NC * NS
EPW = N_EDGES // NW
C = 80
NCHUNK = EPW // C
G = C // L
UNROLL = 8


def _body(z_hbm, src_hbm, dst_hbm, out_hbm,
          idx_s_v, idx_d_v, rows_s, rows_d, out_v, sem_s, sem_d):
    cid = lax.axis_index("c")
    sid = lax.axis_index("s")
    wid = sid * NC + cid
    ebase = wid * EPW

    pltpu.sync_copy(src_hbm.at[pl.ds(ebase, EPW)], idx_s_v)
    pltpu.sync_copy(dst_hbm.at[pl.ds(ebase, EPW)], idx_d_v)

    lanes = lax.iota(jnp.int32, L)
    row_ids = [lanes + k * L for k in range(G)]

    def chunk(g, carry):
        off = g * C
        cp_s = pltpu.async_copy(z_hbm.at[idx_s_v.at[pl.ds(off, C)]], rows_s, sem_s)
        cp_d = pltpu.async_copy(z_hbm.at[idx_d_v.at[pl.ds(off, C)]], rows_d, sem_d)
        cp_s.wait()
        cp_d.wait()

        def dstep(j, accs):
            d0 = j * UNROLL
            col0 = jnp.zeros((L,), jnp.int32) + d0
            out = []
            for k in range(G):
                acc = accs[k]
                for u in range(UNROLL):
                    col = col0 + u
                    s = plsc.load_gather(rows_s, [row_ids[k], col])
                    t = plsc.load_gather(rows_d, [row_ids[k], col])
                    acc = acc + s * t
                out.append(acc)
            return tuple(out)

        zero = jnp.zeros((L,), jnp.float32)
        accs = lax.fori_loop(0, D_FEAT // UNROLL, dstep, (zero,) * G)
        for k in range(G):
            val = accs[k]
            res = 1.0 / (1.0 + jnp.exp(-val))
            out_v[pl.ds(off + k * L, L)] = res
        return carry

    lax.fori_loop(0, NCHUNK, chunk, 0)
    pltpu.sync_copy(out_v, out_hbm.at[pl.ds(ebase, EPW)])


_mesh = plsc.VectorSubcoreMesh(
    core_axis_name="c", subcore_axis_name="s", num_cores=NC, num_subcores=NS)

_call = functools.partial(
    pl.kernel,
    out_type=jax.ShapeDtypeStruct((N_EDGES,), jnp.float32),
    mesh=_mesh,
    scratch_types=[
        pltpu.VMEM((EPW,), jnp.int32),
        pltpu.VMEM((EPW,), jnp.int32),
        pltpu.VMEM((C, D_FEAT), jnp.float32),
        pltpu.VMEM((C, D_FEAT), jnp.float32),
        pltpu.VMEM((EPW,), jnp.float32),
        pltpu.SemaphoreType.DMA,
        pltpu.SemaphoreType.DMA,
    ],
)(_body)


def kernel(z, edge_index):
    src = edge_index[0]
    dst = edge_index[1]
    return _call(z, src, dst)


# speedup vs baseline: 1.2072x; 1.2072x over previous
import functools

import jax
import jax.numpy as jnp
from jax import lax
from jax.experimental import pallas as pl
from jax.experimental.pallas import tpu as pltpu
from jax.experimental.pallas import tpu_sc as plsc

N_NODES = 10000
N_EDGES = 320000
D_FEAT = 128

NC = 2
NS = 16
L = 16
NW = NC * NS
EPW = N_EDGES // NW
C = 80
NCHUNK = EPW // C
G = C // L
UNROLL = 8


def _body(z_hbm, src_hbm, dst_hbm, out_hbm,
          idx_s_v, idx_d_v, rows_s, rows_d, out_v, sem_s, sem_d):
    cid = lax.axis_index("c")
    sid = lax.axis_index("s")
    wid = sid * NC + cid
    ebase = wid * EPW

    pltpu.sync_copy(src_hbm.at[pl.ds(ebase, EPW)], idx_s_v)
    pltpu.sync_copy(dst_hbm.at[pl.ds(ebase, EPW)], idx_d_v)

    lanes = lax.iota(jnp.int32, L)
    row_ids = [lanes + k * L for k in range(G)]

    def chunk(g, carry):
        off = g * C
        cp_s = pltpu.async_copy(z_hbm.at[idx_s_v.at[pl.ds(off, C)]], rows_s, sem_s)
        cp_d = pltpu.async_copy(z_hbm.at[idx_d_v.at[pl.ds(off, C)]], rows_d, sem_d)
        cp_s.wait()
        cp_d.wait()

        def dstep(j, accs):
            d0 = j * UNROLL
            col0 = jnp.zeros((L,), jnp.int32) + d0
            out = []
            for k in range(G):
                acc = accs[k]
                for u in range(UNROLL):
                    col = col0 + u
                    s = plsc.load_gather(rows_s, [row_ids[k], col])
                    t = plsc.load_gather(rows_d, [row_ids[k], col])
                    acc = acc + s * t
                out.append(acc)
            return tuple(out)

        zero = jnp.zeros((L,), jnp.float32)
        accs = lax.fori_loop(0, D_FEAT // UNROLL, dstep, (zero,) * G)
        for k in range(G):
            val = accs[k]
            res = 1.0 / (1.0 + jnp.exp(-val))
            out_v[pl.ds(off + k * L, L)] = res
        return carry

    lax.fori_loop(0, NCHUNK, chunk, 0)
    pltpu.sync_copy(out_v, out_hbm.at[pl.ds(ebase, EPW)])


_mesh = plsc.VectorSubcoreMesh(
    core_axis_name="c", subcore_axis_name="s", num_cores=NC, num_subcores=NS)

_call = functools.partial(
    pl.kernel,
    out_type=jax.ShapeDtypeStruct((N_EDGES,), jnp.float32),
    mesh=_mesh,
    scratch_types=[
        pltpu.VMEM((EPW,), jnp.int32),
        pltpu.VMEM((EPW,), jnp.int32),
        pltpu.VMEM((C, D_FEAT), jnp.float32),
        pltpu.VMEM((C, D_FEAT), jnp.float32),
        pltpu.VMEM((EPW,), jnp.float32),
        pltpu.SemaphoreType.DMA,
        pltpu.SemaphoreType.DMA,
    ],
    compiler_params=pltpu.CompilerParams(needs_layout_passes=False),
)(_body)


def kernel(z, edge_index):
    src = edge_index[0]
    dst = edge_index[1]
    return _call(z, src, dst)


# speedup vs baseline: 1.2552x; 1.0398x over previous
import functools

import jax
import jax.numpy as jnp
from jax import lax
from jax.experimental import pallas as pl
from jax.experimental.pallas import tpu as pltpu
from jax.experimental.pallas import tpu_sc as plsc

N_NODES = 10000
N_EDGES = 320000
D_FEAT = 128

NC = 2
NS = 16
L = 16
NW = NC * NS
EPW = N_EDGES // NW
C = 80
NCHUNK = EPW // C
G = C // L
UNROLL = 8


def _body(z_hbm, src_hbm, dst_hbm, out_hbm,
          z_sh, idx_s_v, idx_d_v, rows_s, rows_d, out_v, sem_s, sem_d):
    cid = lax.axis_index("c")
    sid = lax.axis_index("s")
    wid = sid * NC + cid
    ebase = wid * EPW

    zrows = 1000

    @pl.when(sid < N_NODES // zrows)
    def _stage():
        pltpu.sync_copy(z_hbm.at[pl.ds(sid * zrows, zrows)],
                        z_sh.at[pl.ds(sid * zrows, zrows)])

    pltpu.sync_copy(src_hbm.at[pl.ds(ebase, EPW)], idx_s_v)
    pltpu.sync_copy(dst_hbm.at[pl.ds(ebase, EPW)], idx_d_v)
    plsc.subcore_barrier()

    lanes = lax.iota(jnp.int32, L)
    row_ids = [lanes + k * L for k in range(G)]

    def chunk(g, carry):
        off = g * C
        cp_s = pltpu.async_copy(z_sh.at[idx_s_v.at[pl.ds(off, C)]], rows_s, sem_s)
        cp_d = pltpu.async_copy(z_sh.at[idx_d_v.at[pl.ds(off, C)]], rows_d, sem_d)
        cp_s.wait()
        cp_d.wait()

        def dstep(j, accs):
            d0 = j * UNROLL
            col0 = jnp.zeros((L,), jnp.int32) + d0
            out = []
            for k in range(G):
                acc = accs[k]
                for u in range(UNROLL):
                    col = col0 + u
                    s = plsc.load_gather(rows_s, [row_ids[k], col])
                    t = plsc.load_gather(rows_d, [row_ids[k], col])
                    acc = acc + s * t
                out.append(acc)
            return tuple(out)

        zero = jnp.zeros((L,), jnp.float32)
        accs = lax.fori_loop(0, D_FEAT // UNROLL, dstep, (zero,) * G)
        for k in range(G):
            val = accs[k]
            res = 1.0 / (1.0 + jnp.exp(-val))
            out_v[pl.ds(off + k * L, L)] = res
        return carry

    lax.fori_loop(0, NCHUNK, chunk, 0)
    pltpu.sync_copy(out_v, out_hbm.at[pl.ds(ebase, EPW)])


_mesh = plsc.VectorSubcoreMesh(
    core_axis_name="c", subcore_axis_name="s", num_cores=NC, num_subcores=NS)

_call = functools.partial(
    pl.kernel,
    out_type=jax.ShapeDtypeStruct((N_EDGES,), jnp.float32),
    mesh=_mesh,
    scratch_types=[
        pltpu.VMEM_SHARED((N_NODES, D_FEAT), jnp.float32),
        pltpu.VMEM((EPW,), jnp.int32),
        pltpu.VMEM((EPW,), jnp.int32),
        pltpu.VMEM((C, D_FEAT), jnp.float32),
        pltpu.VMEM((C, D_FEAT), jnp.float32),
        pltpu.VMEM((EPW,), jnp.float32),
        pltpu.SemaphoreType.DMA,
        pltpu.SemaphoreType.DMA,
    ],
    compiler_params=pltpu.CompilerParams(needs_layout_passes=False),
)(_body)


def kernel(z, edge_index):
    src = edge_index[0]
    dst = edge_index[1]
    return _call(z, src, dst)


# speedup vs baseline: 6.6685x; 5.3126x over previous
import functools

import jax
import jax.numpy as jnp
from jax import lax
from jax.experimental import pallas as pl
from jax.experimental.pallas import tpu as pltpu
from jax.experimental.pallas import tpu_sc as plsc

N_NODES = 10000
N_EDGES = 320000
D_FEAT = 128

NC = 2
NS = 16
L = 16
NW = NC * NS
EPW = N_EDGES // NW
C = 80
NCHUNK = EPW // C
G = C // L
UNROLL = 8


def _body(z_hbm, src_hbm, dst_hbm, out_hbm,
          z_sh, idx_s_v, idx_d_v, rows_s, rows_d, out_v, sem_s, sem_d):
    cid = lax.axis_index("c")
    sid = lax.axis_index("s")
    wid = sid * NC + cid
    ebase = wid * EPW

    zrows = 1000

    @pl.when(sid < N_NODES // zrows)
    def _stage():
        pltpu.sync_copy(z_hbm.at[pl.ds(sid * zrows, zrows)],
                        z_sh.at[pl.ds(sid * zrows, zrows)])

    pltpu.sync_copy(src_hbm.at[pl.ds(ebase, EPW)], idx_s_v)
    pltpu.sync_copy(dst_hbm.at[pl.ds(ebase, EPW)], idx_d_v)
    plsc.subcore_barrier()

    lanes = lax.iota(jnp.int32, L)
    lane_eq = [lanes == u for u in range(L)]

    def chunk(g, carry):
        off = g * C
        cp_s = pltpu.async_copy(z_sh.at[idx_s_v.at[pl.ds(off, C)]], rows_s, sem_s)
        cp_d = pltpu.async_copy(z_sh.at[idx_d_v.at[pl.ds(off, C)]], rows_d, sem_d)
        cp_s.wait()
        cp_d.wait()

        def estep(blk, _):
            e_base = blk * L

            def one_edge(u, res):
                e = e_base + u
                acc = rows_s[e, pl.ds(0, L)] * rows_d[e, pl.ds(0, L)]
                for j in range(1, D_FEAT // L):
                    acc = acc + (rows_s[e, pl.ds(j * L, L)]
                                 * rows_d[e, pl.ds(j * L, L)])
                return jnp.where(lanes == u, jnp.sum(acc), res)

            res = lax.fori_loop(0, L, one_edge, jnp.zeros((L,), jnp.float32))
            out_v[pl.ds(off + e_base, L)] = res
            return _

        lax.fori_loop(0, G, estep, 0)
        return carry

    lax.fori_loop(0, NCHUNK, chunk, 0)

    def sig(i, _):
        v = out_v[pl.ds(i * L, L)]
        out_v[pl.ds(i * L, L)] = 1.0 / (1.0 + jnp.exp(-v))
        return _

    lax.fori_loop(0, EPW // L, sig, 0)
    pltpu.sync_copy(out_v, out_hbm.at[pl.ds(ebase, EPW)])


_mesh = plsc.VectorSubcoreMesh(
    core_axis_name="c", subcore_axis_name="s", num_cores=NC, num_subcores=NS)

_call = functools.partial(
    pl.kernel,
    out_type=jax.ShapeDtypeStruct((N_EDGES,), jnp.float32),
    mesh=_mesh,
    scratch_types=[
        pltpu.VMEM_SHARED((N_NODES, D_FEAT), jnp.float32),
        pltpu.VMEM((EPW,), jnp.int32),
        pltpu.VMEM((EPW,), jnp.int32),
        pltpu.VMEM((C, D_FEAT), jnp.float32),
        pltpu.VMEM((C, D_FEAT), jnp.float32),
        pltpu.VMEM((EPW,), jnp.float32),
        pltpu.SemaphoreType.DMA,
        pltpu.SemaphoreType.DMA,
    ],
    compiler_params=pltpu.CompilerParams(needs_layout_passes=False),
)(_body)


def kernel(z, edge_index):
    src = edge_index[0]
    dst = edge_index[1]
    return _call(z, src, dst)


# speedup vs baseline: 11.1510x; 1.6722x over previous
import functools

import jax
import jax.numpy as jnp
from jax import lax
from jax.experimental import pallas as pl
from jax.experimental.pallas import tpu as pltpu
from jax.experimental.pallas import tpu_sc as plsc

N_NODES = 10000
N_EDGES = 320000
D_FEAT = 128

NC = 2
NS = 16
L = 16
NW = NC * NS
EPW = N_EDGES // NW
C = 80
NCHUNK = EPW // C
G = C // L


def _body(z_hbm, src_hbm, dst_hbm, out_hbm,
          z_sh, idx_s0, idx_d0, idx_s1, idx_d1,
          rows_s0, rows_d0, rows_s1, rows_d1, out0, out1,
          sem_is0, sem_id0, sem_is1, sem_id1,
          sem_s0, sem_d0, sem_s1, sem_d1, sem_o0, sem_o1):
    cid = lax.axis_index("c")
    sid = lax.axis_index("s")
    wid = sid * NC + cid
    ebase = wid * EPW

    zrows = 1000

    @pl.when(sid < N_NODES // zrows)
    def _stage():
        pltpu.sync_copy(z_hbm.at[pl.ds(sid * zrows, zrows)],
                        z_sh.at[pl.ds(sid * zrows, zrows)])

    plsc.subcore_barrier()

    lanes = lax.iota(jnp.int32, L)
    bufs = ((idx_s0, idx_d0, sem_is0, sem_id0,
             rows_s0, rows_d0, sem_s0, sem_d0, out0, sem_o0),
            (idx_s1, idx_d1, sem_is1, sem_id1,
             rows_s1, rows_d1, sem_s1, sem_d1, out1, sem_o1))

    def start_idx(g, b):
        xs, xd, sis, sid_, _, _, _, _, _, _ = bufs[b]
        off = ebase + g * C
        pltpu.async_copy(src_hbm.at[pl.ds(off, C)], xs, sis)
        pltpu.async_copy(dst_hbm.at[pl.ds(off, C)], xd, sid_)

    def wait_idx(b):
        xs, xd, sis, sid_, _, _, _, _, _, _ = bufs[b]
        pltpu.make_async_copy(src_hbm.at[pl.ds(0, C)], xs, sis).wait()
        pltpu.make_async_copy(src_hbm.at[pl.ds(0, C)], xd, sid_).wait()

    def start_rows(b):
        xs, xd, _, _, rs, rd, ss, sd, _, _ = bufs[b]
        pltpu.async_copy(z_sh.at[xs], rs, ss)
        pltpu.async_copy(z_sh.at[xd], rd, sd)

    def wait_rows(b):
        _, _, _, _, rs, rd, ss, sd, _, _ = bufs[b]
        pltpu.make_async_copy(z_hbm.at[pl.ds(0, C)], rs, ss).wait()
        pltpu.make_async_copy(z_hbm.at[pl.ds(0, C)], rd, sd).wait()

    def compute(g, b):
        _, _, _, _, rs, rd, _, _, ob, so = bufs[b]

        @pl.when(g >= 2)
        def _drain():
            pltpu.make_async_copy(ob, out_hbm.at[pl.ds(0, C)], so).wait()

        def estep(blk, _):
            e_base = blk * L

            def one_edge(u, res):
                e = e_base + u
                acc = rs[e, pl.ds(0, L)] * rd[e, pl.ds(0, L)]
                for j in range(1, D_FEAT // L):
                    acc = acc + (rs[e, pl.ds(j * L, L)]
                                 * rd[e, pl.ds(j * L, L)])
                return jnp.where(lanes == u, jnp.sum(acc), res)

            res = lax.fori_loop(0, L, one_edge, jnp.zeros((L,), jnp.float32))
            ob[pl.ds(e_base, L)] = 1.0 / (1.0 + jnp.exp(-res))
            return _

        lax.fori_loop(0, G, estep, 0)
        pltpu.async_copy(ob, out_hbm.at[pl.ds(ebase + g * C, C)], so)

    start_idx(0, 0)
    wait_idx(0)
    start_rows(0)
    start_idx(1, 1)

    def pair(i, carry):
        g = i * 2
        wait_idx(1)
        start_rows(1)
        wait_rows(0)
        start_idx(g + 2, 0)
        compute(g, 0)
        wait_idx(0)
        start_rows(0)
        wait_rows(1)
        start_idx(jnp.minimum(g + 3, NCHUNK - 1), 1)
        compute(g + 1, 1)
        return carry

    lax.fori_loop(0, (NCHUNK - 1) // 2, pair, 0)
    wait_rows(0)
    compute(NCHUNK - 1, 0)
    wait_idx(1)

    pltpu.make_async_copy(out0, out_hbm.at[pl.ds(0, C)], sem_o0).wait()
    pltpu.make_async_copy(out1, out_hbm.at[pl.ds(0, C)], sem_o1).wait()


_mesh = plsc.VectorSubcoreMesh(
    core_axis_name="c", subcore_axis_name="s", num_cores=NC, num_subcores=NS)

_call = functools.partial(
    pl.kernel,
    out_type=jax.ShapeDtypeStruct((N_EDGES,), jnp.float32),
    mesh=_mesh,
    scratch_types=[
        pltpu.VMEM_SHARED((N_NODES, D_FEAT), jnp.float32),
        pltpu.VMEM((C,), jnp.int32),
        pltpu.VMEM((C,), jnp.int32),
        pltpu.VMEM((C,), jnp.int32),
        pltpu.VMEM((C,), jnp.int32),
        pltpu.VMEM((C, D_FEAT), jnp.float32),
        pltpu.VMEM((C, D_FEAT), jnp.float32),
        pltpu.VMEM((C, D_FEAT), jnp.float32),
        pltpu.VMEM((C, D_FEAT), jnp.float32),
        pltpu.VMEM((C,), jnp.float32),
        pltpu.VMEM((C,), jnp.float32),
        pltpu.SemaphoreType.DMA,
        pltpu.SemaphoreType.DMA,
        pltpu.SemaphoreType.DMA,
        pltpu.SemaphoreType.DMA,
        pltpu.SemaphoreType.DMA,
        pltpu.SemaphoreType.DMA,
        pltpu.SemaphoreType.DMA,
        pltpu.SemaphoreType.DMA,
        pltpu.SemaphoreType.DMA,
        pltpu.SemaphoreType.DMA,
    ],
    compiler_params=pltpu.CompilerParams(needs_layout_passes=False),
)(_body)


def kernel(z, edge_index):
    src = edge_index[0]
    dst = edge_index[1]
    return _call(z, src, dst)
